# baseline (speedup 1.0000x reference)
import functools

import jax
import jax.numpy as jnp
from jax import lax
from jax.experimental import pallas as pl
from jax.experimental.pallas import tpu as pltpu
from jax.experimental.pallas import tpu_sc as plsc

NUM_NODES = 100000
EMBED_DIM = 64
BATCH = 16384
COMB_LEN = 20

NC = 2
NS = 16
NW = NC * NS
B_PER_W = BATCH // NW
CB = 32
NCHUNK = B_PER_W // CB
ROWS_PER_CHUNK = CB * COMB_LEN


def _sc_body(comb_hbm, weight_hbm, out_hbm, idx_v, rows_v, scr_v, outc_v,
             gsem0, gsem1, osem):
    wid = lax.axis_index("s") * NC + lax.axis_index("c")
    gsems = (gsem0, gsem1)
    lane = lax.iota(jnp.int32, 16)

    pltpu.sync_copy(comb_hbm.at[pl.ds(wid * B_PER_W, B_PER_W)], idx_v)

    def fire(buf, sem, c):
        for e in range(CB):
            pltpu.async_copy(
                weight_hbm.at[idx_v.at[c * CB + e]],
                rows_v.at[buf, pl.ds(e * COMB_LEN, COMB_LEN)],
                sem,
            )

    def drain_rows(buf, sem):
        pltpu.make_async_copy(
            weight_hbm.at[pl.ds(0, ROWS_PER_CHUNK)], rows_v.at[buf], sem
        ).wait()

    def drain_out(buf):
        pltpu.make_async_copy(
            outc_v.at[pl.ds(buf * CB, CB)], out_hbm.at[pl.ds(0, CB)], osem
        ).wait()

    def compute(buf, c, need_drain):
        drain_rows(buf, gsems[buf])

        def prod_body(e, _):
            r0 = e * COMB_LEN
            acc = [rows_v[buf, r0, pl.ds(k * 16, 16)] for k in range(4)]
            for l in range(1, COMB_LEN):
                for k in range(4):
                    acc[k] = acc[k] * rows_v[buf, r0 + l, pl.ds(k * 16, 16)]
            s = (acc[0] + acc[1]) + (acc[2] + acc[3])
            scr_v[pl.ds(8 + e * 16, 16)] = s
            return ()

        lax.fori_loop(0, CB, prod_body, ())

        for s in (8, 4, 2, 1):
            def round_body(e, _, s=s):
                base = 8 + e * 16
                a = scr_v[pl.ds(base, 16)]
                b = scr_v[pl.ds(base + (s - 2 * (e & s)), 16)]
                scr_v[pl.ds(base, 16)] = a + b
                return ()

            lax.fori_loop(0, CB, round_body, ())

        @pl.when(need_drain)
        def _():
            drain_out(buf)

        for g in range(CB // 16):
            def merge_body(e, res, g=g):
                v = scr_v[pl.ds(8 + (g * 16 + e) * 16, 16)]
                return jnp.where(lane == e, v, res)

            res = lax.fori_loop(0, 16, merge_body,
                                jnp.zeros((16,), jnp.float32))
            outc_v[pl.ds(buf * CB + g * 16, 16)] = res

        pltpu.async_copy(
            outc_v.at[pl.ds(buf * CB, CB)],
            out_hbm.at[pl.ds(wid * B_PER_W + c * CB, CB)],
            osem,
        )

    fire(0, gsem0, 0)

    def pair_body(i, _):
        c0 = i * 2
        fire(1, gsem1, c0 + 1)
        compute(0, c0, i > 0)

        @pl.when(i < NCHUNK // 2 - 1)
        def _():
            fire(0, gsem0, c0 + 2)

        compute(1, c0 + 1, i > 0)
        return ()

    lax.fori_loop(0, NCHUNK // 2, pair_body, ())
    drain_out(0)
    drain_out(1)


@jax.jit
def _hyper_embed(comb, weight):
    mesh = plsc.VectorSubcoreMesh(core_axis_name="c", subcore_axis_name="s")
    sc = functools.partial(
        pl.kernel,
        mesh=mesh,
        compiler_params=pltpu.CompilerParams(use_tc_tiling_on_sc=False),
        out_type=jax.ShapeDtypeStruct((BATCH,), jnp.float32),
        scratch_types=[
            pltpu.VMEM((B_PER_W, COMB_LEN), jnp.int32),
            pltpu.VMEM((2, ROWS_PER_CHUNK, EMBED_DIM), jnp.float32),
            pltpu.VMEM((8 + CB * 16 + 16,), jnp.float32),
            pltpu.VMEM((2 * CB,), jnp.float32),
            pltpu.SemaphoreType.DMA,
            pltpu.SemaphoreType.DMA,
            pltpu.SemaphoreType.DMA,
        ],
    )(_sc_body)
    return sc(comb, weight)


def kernel(combinations, weight):
    return _hyper_embed(combinations.astype(jnp.int32), weight)


# speedup vs baseline: 1.0330x; 1.0330x over previous
import functools

import jax
import jax.numpy as jnp
from jax import lax
from jax.experimental import pallas as pl
from jax.experimental.pallas import tpu as pltpu
from jax.experimental.pallas import tpu_sc as plsc

NUM_NODES = 100000
EMBED_DIM = 64
BATCH = 16384
COMB_LEN = 20

NC = 2
NS = 16
NW = NC * NS
B_PER_W = BATCH // NW
CB = 32
NCHUNK = B_PER_W // CB
ROWS_PER_CHUNK = CB * COMB_LEN
NGATHER = ROWS_PER_CHUNK // 128
CPAD = ROWS_PER_CHUNK + 16


def _sc_body(comb_hbm, weight_hbm, out_hbm, idx_v, cmp_v, rows_v, scr_v,
             outc_v, isem, gsem0, gsem1, osem):
    wid = lax.axis_index("s") * NC + lax.axis_index("c")
    gsems = (gsem0, gsem1)
    lane = lax.iota(jnp.int32, 16)

    def fire_idx(buf, c):
        pltpu.async_copy(
            comb_hbm.at[pl.ds(wid * B_PER_W + c * CB, CB)], idx_v.at[buf], isem
        )

    def drain_idx(buf):
        pltpu.make_async_copy(
            comb_hbm.at[pl.ds(0, CB)], idx_v.at[buf], isem
        ).wait()

    def compact(buf):
        def body(e, _):
            v0 = idx_v[buf, e, pl.ds(0, 16)]
            v1 = idx_v[buf, e, pl.ds(16, 16)]
            cmp_v[pl.ds(buf * CPAD + e * COMB_LEN, 16)] = v0
            cmp_v[pl.ds(buf * CPAD + e * COMB_LEN + 16, 16)] = v1
            return ()

        lax.fori_loop(0, CB, body, ())

    def fire_rows(buf):
        for j in range(NGATHER):
            pltpu.async_copy(
                weight_hbm.at[cmp_v.at[pl.ds(buf * CPAD + j * 128, 128)]],
                rows_v.at[buf, pl.ds(j * 128, 128)],
                gsems[buf],
            )

    def drain_rows(buf):
        pltpu.make_async_copy(
            weight_hbm.at[pl.ds(0, ROWS_PER_CHUNK)], rows_v.at[buf], gsems[buf]
        ).wait()

    def drain_out(buf):
        pltpu.make_async_copy(
            outc_v.at[pl.ds(buf * CB, CB)], out_hbm.at[pl.ds(0, CB)], osem
        ).wait()

    def compute(buf, c, need_drain):
        drain_rows(buf)

        def prod_body(e, _):
            r0 = e * COMB_LEN
            acc = [rows_v[buf, r0, pl.ds(k * 16, 16)] for k in range(4)]
            for l in range(1, COMB_LEN):
                for k in range(4):
                    acc[k] = acc[k] * rows_v[buf, r0 + l, pl.ds(k * 16, 16)]
            s = (acc[0] + acc[1]) + (acc[2] + acc[3])
            scr_v[pl.ds(8 + e * 16, 16)] = s
            return ()

        lax.fori_loop(0, CB, prod_body, ())

        for s in (8, 4, 2, 1):
            def round_body(e, _, s=s):
                base = 8 + e * 16
                a = scr_v[pl.ds(base, 16)]
                b = scr_v[pl.ds(base + (s - 2 * (e & s)), 16)]
                scr_v[pl.ds(base, 16)] = a + b
                return ()

            lax.fori_loop(0, CB, round_body, ())

        @pl.when(need_drain)
        def _():
            drain_out(buf)

        for g in range(CB // 16):
            def merge_body(e, res, g=g):
                v = scr_v[pl.ds(8 + (g * 16 + e) * 16, 16)]
                return jnp.where(lane == e, v, res)

            res = lax.fori_loop(0, 16, merge_body,
                                jnp.zeros((16,), jnp.float32))
            outc_v[pl.ds(buf * CB + g * 16, 16)] = res

        pltpu.async_copy(
            outc_v.at[pl.ds(buf * CB, CB)],
            out_hbm.at[pl.ds(wid * B_PER_W + c * CB, CB)],
            osem,
        )

    fire_idx(0, 0)
    fire_idx(1, 1)
    drain_idx(0)
    compact(0)
    fire_rows(0)
    fire_idx(0, 2)

    def pair_body(i, _):
        c0 = i * 2
        drain_idx(1)
        compact(1)
        fire_rows(1)

        @pl.when(i < NCHUNK // 2 - 1)
        def _():
            fire_idx(1, c0 + 3)

        compute(0, c0, i > 0)

        @pl.when(i < NCHUNK // 2 - 1)
        def _():
            drain_idx(0)
            compact(0)
            fire_rows(0)

        @pl.when(i < NCHUNK // 2 - 2)
        def _():
            fire_idx(0, c0 + 4)

        compute(1, c0 + 1, i > 0)
        return ()

    lax.fori_loop(0, NCHUNK // 2, pair_body, ())
    drain_out(0)
    drain_out(1)


@jax.jit
def _hyper_embed(comb_p, weight):
    mesh = plsc.VectorSubcoreMesh(core_axis_name="c", subcore_axis_name="s")
    sc = functools.partial(
        pl.kernel,
        mesh=mesh,
        compiler_params=pltpu.CompilerParams(use_tc_tiling_on_sc=False),
        out_type=jax.ShapeDtypeStruct((BATCH,), jnp.float32),
        scratch_types=[
            pltpu.VMEM((2, CB, 128), jnp.int32),
            pltpu.VMEM((2 * CPAD,), jnp.int32),
            pltpu.VMEM((2, ROWS_PER_CHUNK, EMBED_DIM), jnp.float32),
            pltpu.VMEM((8 + CB * 16 + 16,), jnp.float32),
            pltpu.VMEM((2 * CB,), jnp.float32),
            pltpu.SemaphoreType.DMA,
            pltpu.SemaphoreType.DMA,
            pltpu.SemaphoreType.DMA,
            pltpu.SemaphoreType.DMA,
        ],
    )(_sc_body)
    return sc(comb_p, weight)


def kernel(combinations, weight):
    comb_p = jnp.pad(combinations.astype(jnp.int32),
                     ((0, 0), (0, 128 - COMB_LEN)))
    return _hyper_embed(comb_p, weight)
